# Initial kernel scaffold; baseline (speedup 1.0000x reference)
import jax
import jax.numpy as jnp
from jax.experimental import pallas as pl


def kernel(x, table, W_c, b_c, W_ih_f, W_hh_f, b_ih_f, b_hh_f, W_ih_b, W_hh_b, b_ih_b, b_hh_b):
    raise NotImplementedError("write your pallas kernel here")


# speedup vs baseline: 11.2871x; 11.2871x over previous
import functools

import jax
import jax.numpy as jnp
from jax import lax
from jax.experimental import pallas as pl
from jax.experimental.pallas import tpu as pltpu
from jax.experimental.pallas import tpu_sc as plsc

VOCAB = 1000000
EMB = 128
ENC = 128
HID = 128
B = 1024
L = 50
N_ROWS = B * L


_CH = 80


def _sc_gather(table, idx2d, n_ch, nw, num_cores):
    mesh = plsc.VectorSubcoreMesh(core_axis_name="c", subcore_axis_name="s")
    b_per_w = n_ch * _CH

    @functools.partial(
        pl.kernel,
        out_type=jax.ShapeDtypeStruct((N_ROWS, EMB), jnp.float32),
        mesh=mesh,
        scratch_types=[
            pltpu.VMEM((n_ch, _CH), jnp.int32),
            pltpu.VMEM((_CH, EMB), jnp.float32),
            pltpu.VMEM((_CH, EMB), jnp.float32),
            pltpu.SemaphoreType.DMA,
            pltpu.SemaphoreType.DMA,
            pltpu.SemaphoreType.DMA,
            pltpu.SemaphoreType.DMA,
        ],
    )
    def k(table_hbm, idx_hbm, out_hbm, idx_v, rows0, rows1, g0, g1, o0, o1):
        wid = lax.axis_index("s") * num_cores + lax.axis_index("c")
        base = wid * b_per_w
        pltpu.sync_copy(idx_hbm.at[wid], idx_v)
        rows = (rows0, rows1)
        gsem = (g0, g1)
        osem = (o0, o1)
        gh = [None, None]
        oh = [None, None]
        for j in range(n_ch + 1):
            s = j % 2
            if j < n_ch:
                if oh[s] is not None:
                    oh[s].wait()
                    oh[s] = None
                gh[s] = pltpu.async_copy(
                    table_hbm.at[idx_v.at[j]], rows[s], gsem[s]
                )
            if j >= 1:
                p = (j - 1) % 2
                gh[p].wait()
                oh[p] = pltpu.async_copy(
                    rows[p], out_hbm.at[pl.ds(base + (j - 1) * _CH, _CH)], osem[p]
                )
        for p in range(2):
            if oh[p] is not None:
                oh[p].wait()

    return k(table, idx2d)


def _rnn_kernel(emb_ref, wc_ref, bc_ref, wif_ref, bif_ref, whf_ref, bhf_ref,
                wib_ref, bib_ref, whb_ref, bhb_ref, out_ref,
                hf_ref, hb_ref, mf_ref, mb_ref):
    f32 = jnp.float32
    wc = wc_ref[...]
    a_f = lax.dot_general(wc, wif_ref[...], (((0,), (1,)), ((), ())),
                          preferred_element_type=f32)
    a_b = lax.dot_general(wc, wib_ref[...], (((0,), (1,)), ((), ())),
                          preferred_element_type=f32)
    c_f = lax.dot_general(bc_ref[...], wif_ref[...], (((1,), (1,)), ((), ())),
                          preferred_element_type=f32) + bif_ref[...]
    c_b = lax.dot_general(bc_ref[...], wib_ref[...], (((1,), (1,)), ((), ())),
                          preferred_element_type=f32) + bib_ref[...]
    whf = whf_ref[...]
    whb = whb_ref[...]
    bhf = bhf_ref[...]
    bhb = bhb_ref[...]

    hf_ref[...] = jnp.zeros((B, HID), f32)
    hb_ref[...] = jnp.zeros((B, HID), f32)
    mf_ref[...] = jnp.full((B, HID), -jnp.inf, f32)
    mb_ref[...] = jnp.full((B, HID), -jnp.inf, f32)

    def gru_step(e, h, a, c, wh, bh):
        gi = lax.dot_general(e, a, (((1,), (0,)), ((), ())),
                             preferred_element_type=f32) + c
        gh = lax.dot_general(h, wh, (((1,), (1,)), ((), ())),
                             preferred_element_type=f32) + bh
        r = jax.nn.sigmoid(gi[:, :HID] + gh[:, :HID])
        z = jax.nn.sigmoid(gi[:, HID:2 * HID] + gh[:, HID:2 * HID])
        n = jnp.tanh(gi[:, 2 * HID:] + r * gh[:, 2 * HID:])
        return (1.0 - z) * n + z * h

    def step(t, _):
        e_f = emb_ref[t]
        e_b = emb_ref[L - 1 - t]
        h_f = gru_step(e_f, hf_ref[...], a_f, c_f, whf, bhf)
        h_b = gru_step(e_b, hb_ref[...], a_b, c_b, whb, bhb)
        hf_ref[...] = h_f
        hb_ref[...] = h_b
        mf_ref[...] = jnp.maximum(mf_ref[...], h_f)
        mb_ref[...] = jnp.maximum(mb_ref[...], h_b)
        return 0

    lax.fori_loop(0, L, step, 0)
    out_ref[:, :HID] = mf_ref[...]
    out_ref[:, HID:] = mb_ref[...]


def _tc_rnn(emb, wc, bc, wif, bif, whf, bhf, wib, bib, whb, bhb):
    return pl.pallas_call(
        _rnn_kernel,
        out_shape=jax.ShapeDtypeStruct((B, 2 * HID), jnp.float32),
        scratch_shapes=[
            pltpu.VMEM((B, HID), jnp.float32),
            pltpu.VMEM((B, HID), jnp.float32),
            pltpu.VMEM((B, HID), jnp.float32),
            pltpu.VMEM((B, HID), jnp.float32),
        ],
    )(emb, wc, bc, wif, bif, whf, bhf, wib, bib, whb, bhb)


def kernel(x, table, W_c, b_c, W_ih_f, W_hh_f, b_ih_f, b_hh_f,
           W_ih_b, W_hh_b, b_ih_b, b_hh_b):
    info = plsc.get_sparse_core_info()
    nw = info.num_cores * info.num_subcores
    n_ch = N_ROWS // (nw * _CH)
    idx2d = jnp.transpose(x, (1, 0)).reshape(nw, n_ch, _CH)
    emb = _sc_gather(table, idx2d, n_ch, nw, info.num_cores)
    emb = emb.reshape(L, B, EMB)
    return _tc_rnn(
        emb, W_c, b_c.reshape(1, ENC),
        W_ih_f, b_ih_f.reshape(1, 3 * HID), W_hh_f, b_hh_f.reshape(1, 3 * HID),
        W_ih_b, b_ih_b.reshape(1, 3 * HID), W_hh_b, b_hh_b.reshape(1, 3 * HID),
    )
